# Initial kernel scaffold; baseline (speedup 1.0000x reference)
import jax
import jax.numpy as jnp
from jax.experimental import pallas as pl


def kernel(ground_atoms, clause_weight):
    raise NotImplementedError("write your pallas kernel here")


# speedup vs baseline: 1.9542x; 1.9542x over previous
import numpy as np
import jax
import jax.numpy as jnp
from jax.experimental import pallas as pl
from jax.experimental.pallas import tpu as pltpu

_NUM_P = 256
_NUM_L = 8
_BATCH = 65536
_IDX = np.array([0, 17, 42, 100, 128, 200, 255, 60], dtype=np.int32)
_SGN = np.array([-1.0, 1.0, -1.0, 1.0, -1.0, 1.0, -1.0, 1.0], dtype=np.float32)
_MIN_W = 0.0
_MAX_W = 500.0

_G_SIGNED = np.zeros((_NUM_P, _NUM_L), dtype=np.float32)
_G_SIGNED[_IDX, np.arange(_NUM_L)] = _SGN
_S_SCAT = np.zeros((_NUM_L, _NUM_P), dtype=np.float32)
_S_SCAT[np.arange(_NUM_L), _IDX] = 1.0

_BM = 2048


def _body(w_ref, x_ref, g_ref, s_ref, out_ref, delta_ref):
    x = x_ref[...]
    g = g_ref[...]
    z = jnp.dot(x, g, preferred_element_type=jnp.float32)
    m = jnp.max(z, axis=-1, keepdims=True)
    e = jnp.exp(z - m)
    ssum = jnp.sum(e, axis=-1, keepdims=True)
    w = jnp.clip(w_ref[0], _MIN_W, _MAX_W)
    sgn = jnp.sum(g, axis=0, keepdims=True)
    d = (w * sgn) * (e / ssum)
    delta_ref[...] = d
    out_ref[...] = jnp.dot(d, s_ref[...], preferred_element_type=jnp.float32)


def kernel(ground_atoms, clause_weight):
    b = ground_atoms.shape[0]
    grid = (b // _BM,)
    out, delta = pl.pallas_call(
        _body,
        grid=grid,
        in_specs=[
            pl.BlockSpec(memory_space=pltpu.SMEM),
            pl.BlockSpec((_BM, _NUM_P), lambda i: (i, 0)),
            pl.BlockSpec((_NUM_P, _NUM_L), lambda i: (0, 0)),
            pl.BlockSpec((_NUM_L, _NUM_P), lambda i: (0, 0)),
        ],
        out_specs=[
            pl.BlockSpec((_BM, _NUM_P), lambda i: (i, 0)),
            pl.BlockSpec((_BM, _NUM_L), lambda i: (i, 0)),
        ],
        out_shape=[
            jax.ShapeDtypeStruct((b, _NUM_P), jnp.float32),
            jax.ShapeDtypeStruct((b, _NUM_L), jnp.float32),
        ],
        compiler_params=pltpu.CompilerParams(
            dimension_semantics=("arbitrary",),
        ),
    )(
        jnp.reshape(clause_weight.astype(jnp.float32), (1,)),
        ground_atoms,
        jnp.asarray(_G_SIGNED),
        jnp.asarray(_S_SCAT),
    )
    return out, delta


# speedup vs baseline: 2.1466x; 1.0984x over previous
import numpy as np
import jax
import jax.numpy as jnp
from jax.experimental import pallas as pl
from jax.experimental.pallas import tpu as pltpu

_NUM_P = 256
_NUM_L = 8
_BATCH = 65536
_IDX = np.array([0, 17, 42, 100, 128, 200, 255, 60], dtype=np.int32)
_SGN = np.array([-1.0, 1.0, -1.0, 1.0, -1.0, 1.0, -1.0, 1.0], dtype=np.float32)
_MIN_W = 0.0
_MAX_W = 500.0

_G_SIGNED = np.zeros((_NUM_P, _NUM_L), dtype=np.float32)
_G_SIGNED[_IDX, np.arange(_NUM_L)] = _SGN
_S_SCAT = np.zeros((_NUM_L, _NUM_P), dtype=np.float32)
_S_SCAT[np.arange(_NUM_L), _IDX] = 1.0

_BM = 4096


def _body(w_ref, x_ref, g_ref, s_ref, out_ref, delta_ref):
    x = x_ref[...]
    g = g_ref[...]
    z = jnp.dot(x, g, preferred_element_type=jnp.float32)
    m = jnp.max(z, axis=-1, keepdims=True)
    e = jnp.exp(z - m)
    ssum = jnp.sum(e, axis=-1, keepdims=True)
    w = jnp.clip(w_ref[0], _MIN_W, _MAX_W)
    sgn = jnp.sum(g, axis=0, keepdims=True)
    d = (w * sgn) * (e / ssum)
    delta_ref[...] = d
    out_ref[...] = jnp.dot(d, s_ref[...], preferred_element_type=jnp.float32)


def kernel(ground_atoms, clause_weight):
    b = ground_atoms.shape[0]
    grid = (b // _BM,)
    out, delta = pl.pallas_call(
        _body,
        grid=grid,
        in_specs=[
            pl.BlockSpec(memory_space=pltpu.SMEM),
            pl.BlockSpec((_BM, _NUM_P), lambda i: (i, 0)),
            pl.BlockSpec((_NUM_P, _NUM_L), lambda i: (0, 0)),
            pl.BlockSpec((_NUM_L, _NUM_P), lambda i: (0, 0)),
        ],
        out_specs=[
            pl.BlockSpec((_BM, _NUM_P), lambda i: (i, 0)),
            pl.BlockSpec((_BM, _NUM_L), lambda i: (i, 0)),
        ],
        out_shape=[
            jax.ShapeDtypeStruct((b, _NUM_P), jnp.float32),
            jax.ShapeDtypeStruct((b, _NUM_L), jnp.float32),
        ],
        compiler_params=pltpu.CompilerParams(
            dimension_semantics=("parallel",),
        ),
    )(
        jnp.reshape(clause_weight.astype(jnp.float32), (1,)),
        ground_atoms,
        jnp.asarray(_G_SIGNED),
        jnp.asarray(_S_SCAT),
    )
    return out, delta


# speedup vs baseline: 2.2320x; 1.0398x over previous
import numpy as np
import jax
import jax.numpy as jnp
from jax.experimental import pallas as pl
from jax.experimental.pallas import tpu as pltpu

_NUM_P = 256
_NUM_L = 8
_BATCH = 65536
_IDX = np.array([0, 17, 42, 100, 128, 200, 255, 60], dtype=np.int32)
_SGN = np.array([-1.0, 1.0, -1.0, 1.0, -1.0, 1.0, -1.0, 1.0], dtype=np.float32)
_MIN_W = 0.0
_MAX_W = 500.0

_G_SIGNED = np.zeros((_NUM_P, _NUM_L), dtype=np.float32)
_G_SIGNED[_IDX, np.arange(_NUM_L)] = _SGN
_S_SCAT = np.zeros((_NUM_L, _NUM_P), dtype=np.float32)
_S_SCAT[np.arange(_NUM_L), _IDX] = 1.0

_BM = 8192


def _body(w_ref, x_ref, g_ref, s_ref, out_ref, delta_ref):
    x = x_ref[...]
    g = g_ref[...]
    z = jnp.dot(x, g, preferred_element_type=jnp.float32)
    m = jnp.max(z, axis=-1, keepdims=True)
    e = jnp.exp(z - m)
    ssum = jnp.sum(e, axis=-1, keepdims=True)
    w = jnp.clip(w_ref[0], _MIN_W, _MAX_W)
    sgn = jnp.sum(g, axis=0, keepdims=True)
    d = (w * sgn) * (e / ssum)
    delta_ref[...] = d
    out_ref[...] = jnp.dot(d, s_ref[...], preferred_element_type=jnp.float32)


def kernel(ground_atoms, clause_weight):
    b = ground_atoms.shape[0]
    grid = (b // _BM,)
    out, delta = pl.pallas_call(
        _body,
        grid=grid,
        in_specs=[
            pl.BlockSpec(memory_space=pltpu.SMEM),
            pl.BlockSpec((_BM, _NUM_P), lambda i: (i, 0)),
            pl.BlockSpec((_NUM_P, _NUM_L), lambda i: (0, 0)),
            pl.BlockSpec((_NUM_L, _NUM_P), lambda i: (0, 0)),
        ],
        out_specs=[
            pl.BlockSpec((_BM, _NUM_P), lambda i: (i, 0)),
            pl.BlockSpec((_BM, _NUM_L), lambda i: (i, 0)),
        ],
        out_shape=[
            jax.ShapeDtypeStruct((b, _NUM_P), jnp.float32),
            jax.ShapeDtypeStruct((b, _NUM_L), jnp.float32),
        ],
        compiler_params=pltpu.CompilerParams(
            dimension_semantics=("parallel",),
        ),
    )(
        jnp.reshape(clause_weight.astype(jnp.float32), (1,)),
        ground_atoms,
        jnp.asarray(_G_SIGNED),
        jnp.asarray(_S_SCAT),
    )
    return out, delta
